# Initial kernel scaffold; baseline (speedup 1.0000x reference)
import jax
import jax.numpy as jnp
from jax.experimental import pallas as pl


def kernel(feature, relation_indices, relation_weight, W, b):
    raise NotImplementedError("write your pallas kernel here")


# speedup vs baseline: 2.8631x; 2.8631x over previous
import functools

import jax
import jax.numpy as jnp
from jax import lax
from jax.experimental import pallas as pl
from jax.experimental.pallas import tpu as pltpu
from jax.experimental.pallas import tpu_sc as plsc

N_NODES = 10000
N_PAD = 10240
N_EDGES = 320000
D = 128
NC = 2
NS = 16
NW = NC * NS
EDGES_PER_TILE = N_EDGES // NW
CHUNK = 128
NCHUNK = -(-EDGES_PER_TILE // CHUNK)
NCALL = 4
CPC = -(-NCHUNK // NCALL)
EP_TILE = NCALL * CPC * CHUNK
ROWS_PER_TILE = N_PAD // NS
PAD_ROW = N_PAD - 1
ZR = (128, 128, 128, 128, 128)


def _sc_aggregate(feature, epack, z128):
  mesh = plsc.VectorSubcoreMesh(core_axis_name="c", subcore_axis_name="s")

  @functools.partial(
      pl.kernel,
      mesh=mesh,
      compiler_params=pltpu.CompilerParams(needs_layout_passes=False),
      out_type=[
          jax.ShapeDtypeStruct((NC, N_PAD, D), jnp.float32),
          jax.ShapeDtypeStruct((NW, N_PAD), jnp.float32),
      ],
      scratch_types=[
          pltpu.VMEM((3, CHUNK), jnp.int32),
          pltpu.VMEM((CHUNK, D), jnp.float32),
          pltpu.VMEM((N_PAD,), jnp.float32),
          pltpu.VMEM_SHARED((N_PAD, D), jnp.float32),
          pltpu.SemaphoreType.DMA,
      ],
  )
  def k(feature_hbm, epack_hbm, z128_hbm,
        sums_out, cnts_out, ep, gbuf, hist, sum_acc, sem):
    cid = lax.axis_index("c")
    sid = lax.axis_index("s")
    wid = cid * NS + sid
    row0 = sid * ROWS_PER_TILE

    zv = jnp.zeros((16,), jnp.float32)
    @pl.loop(0, N_PAD // 16)
    def zhist(i):
      hist[pl.ds(i * 16, 16)] = zv

    pltpu.sync_copy(z128_hbm, gbuf)
    r = 0
    for n in ZR:
      pltpu.async_copy(gbuf.at[pl.ds(0, n)],
                       sum_acc.at[pl.ds(row0 + r, n)], sem).wait()
      r += n
    plsc.subcore_barrier()

    idx1 = jnp.full((16,), 1, jnp.int32)
    idx2 = jnp.full((16,), 2, jnp.int32)
    lane0 = lax.iota(jnp.int32, 16) == 0
    one16 = jnp.full((16,), 1.0, jnp.float32)

    for c in range(CPC):
      pltpu.async_copy(epack_hbm.at[wid, c], ep, sem).wait()
      pltpu.async_copy(feature_hbm.at[ep.at[0]], gbuf, sem).wait()

      @pl.loop(0, CHUNK)
      def mul_body(e):
        ev = jnp.full((16,), e, jnp.int32)
        wbits = plsc.load_gather(ep, [idx2, ev])
        w = plsc.bitcast(wbits, jnp.float32)
        dstv = plsc.load_gather(ep, [idx1, ev])
        plsc.addupdate_scatter(hist, [dstv], one16, mask=lane0)
        for j in range(D // 16):
          gbuf[e, pl.ds(j * 16, 16)] = gbuf[e, pl.ds(j * 16, 16)] * w

      pltpu.async_copy(gbuf, sum_acc.at[ep.at[1]], sem, add=True).wait()

    plsc.subcore_barrier()
    r = 0
    for n in ZR:
      pltpu.async_copy(sum_acc.at[pl.ds(row0 + r, n)],
                       gbuf.at[pl.ds(0, n)], sem).wait()
      pltpu.async_copy(gbuf.at[pl.ds(0, n)],
                       sums_out.at[cid, pl.ds(row0 + r, n)], sem).wait()
      r += n
    pltpu.async_copy(hist, cnts_out.at[wid], sem).wait()

  return k(feature, epack, z128)


def _tc_combine(feature, sums, cnt2d, W, b2d):
  R = 2000
  grid = N_NODES // R

  def body(feat_ref, sums_ref, cnt_ref, w_ref, b_ref, out_ref):
    s = sums_ref[0]
    for i in range(1, NCALL * NC):
      s = s + sums_ref[i]
    mean = s / jnp.maximum(cnt_ref[...], 1.0)
    w = w_ref[...]
    out_ref[...] = (
        jnp.dot(feat_ref[...], w[:D], preferred_element_type=jnp.float32)
        + jnp.dot(mean, w[D:], preferred_element_type=jnp.float32)
        + b_ref[...]
    )

  return pl.pallas_call(
      body,
      grid=(grid,),
      in_specs=[
          pl.BlockSpec((R, D), lambda i: (i, 0)),
          pl.BlockSpec((NCALL * NC, R, D), lambda i: (0, i, 0)),
          pl.BlockSpec((R, 1), lambda i: (i, 0)),
          pl.BlockSpec((2 * D, D), lambda i: (0, 0)),
          pl.BlockSpec((1, D), lambda i: (0, 0)),
      ],
      out_specs=pl.BlockSpec((R, D), lambda i: (i, 0)),
      out_shape=jax.ShapeDtypeStruct((N_NODES, D), jnp.float32),
  )(feature, sums, cnt2d, W, b2d)


def kernel(feature, relation_indices, relation_weight, W, b):
  ri = relation_indices.astype(jnp.int32)
  npad = EP_TILE - EDGES_PER_TILE
  dst = jnp.pad(ri[0].reshape(NW, EDGES_PER_TILE), ((0, 0), (0, npad)),
                constant_values=PAD_ROW)
  src = jnp.pad(ri[1].reshape(NW, EDGES_PER_TILE), ((0, 0), (0, npad)))
  wtb = lax.bitcast_convert_type(
      relation_weight.reshape(NW, EDGES_PER_TILE), jnp.int32)
  wtb = jnp.pad(wtb, ((0, 0), (0, npad)))
  epack = jnp.stack([src, dst, wtb], axis=1)
  epack = epack.reshape(NW, 3, NCALL * CPC, CHUNK).transpose(0, 2, 1, 3)
  z128 = jnp.zeros((CHUNK, D), jnp.float32)
  sums_l, cnts_l = [], []
  for kcall in range(NCALL):
    s_k, c_k = _sc_aggregate(
        feature, epack[:, kcall * CPC:(kcall + 1) * CPC], z128)
    sums_l.append(s_k)
    cnts_l.append(c_k)
  sums = jnp.concatenate(sums_l, axis=0)
  cnts = sum(c.sum(axis=0) for c in cnts_l)
  cnt2d = cnts[:N_NODES].reshape(N_NODES, 1)
  return _tc_combine(feature, sums, cnt2d, W, b.reshape(1, D))


# speedup vs baseline: 3.2622x; 1.1394x over previous
import functools

import jax
import jax.numpy as jnp
from jax import lax
from jax.experimental import pallas as pl
from jax.experimental.pallas import tpu as pltpu
from jax.experimental.pallas import tpu_sc as plsc

N_NODES = 10000
N_PAD = 10112
N_EDGES = 320000
D = 128
NC = 2
NS = 16
NW = NC * NS
EDGES_PER_TILE = N_EDGES // NW
CHUNK = 128
NCHUNK = -(-EDGES_PER_TILE // CHUNK)
NCALL = 4
CPC = -(-NCHUNK // NCALL)
EP_TILE = NCALL * CPC * CHUNK
ROWS_PER_TILE = N_PAD // NS
PAD_ROW = N_PAD - 1
ZR = (128, 128, 128, 128, 120)


def _sc_aggregate(feature, epack, z128):
  mesh = plsc.VectorSubcoreMesh(core_axis_name="c", subcore_axis_name="s")

  @functools.partial(
      pl.kernel,
      mesh=mesh,
      compiler_params=pltpu.CompilerParams(needs_layout_passes=False),
      out_type=[
          jax.ShapeDtypeStruct((NC, N_PAD, D), jnp.float32),
          jax.ShapeDtypeStruct((NW, N_PAD), jnp.float32),
      ],
      scratch_types=[
          pltpu.VMEM((3, CHUNK), jnp.int32),
          pltpu.VMEM((3, CHUNK), jnp.int32),
          pltpu.VMEM((CHUNK, D), jnp.float32),
          pltpu.VMEM((CHUNK, D), jnp.float32),
          pltpu.VMEM((N_PAD,), jnp.float32),
          pltpu.VMEM_SHARED((N_PAD, D), jnp.float32),
          pltpu.SemaphoreType.DMA,
          pltpu.SemaphoreType.DMA,
          pltpu.SemaphoreType.DMA,
          pltpu.SemaphoreType.DMA,
          pltpu.SemaphoreType.DMA,
      ],
  )
  def k(feature_hbm, epack_hbm, z128_hbm,
        sums_out, cnts_out, ep0, ep1, gbuf0, gbuf1, hist, sum_acc,
        semS, semG0, semG1, semA0, semA1):
    eps, gbufs = (ep0, ep1), (gbuf0, gbuf1)
    sG, sA = (semG0, semG1), (semA0, semA1)
    gbuf = gbuf0
    sem = semS
    cid = lax.axis_index("c")
    sid = lax.axis_index("s")
    wid = cid * NS + sid
    row0 = sid * ROWS_PER_TILE

    zv = jnp.zeros((16,), jnp.float32)
    @pl.loop(0, N_PAD // 16)
    def zhist(i):
      hist[pl.ds(i * 16, 16)] = zv

    pltpu.sync_copy(z128_hbm, gbuf)
    r = 0
    for n in ZR:
      pltpu.async_copy(gbuf.at[pl.ds(0, n)],
                       sum_acc.at[pl.ds(row0 + r, n)], sem).wait()
      r += n
    plsc.subcore_barrier()

    idx1 = jnp.full((16,), 1, jnp.int32)
    idx2 = jnp.full((16,), 2, jnp.int32)
    lane0 = lax.iota(jnp.int32, 16) == 0
    one16 = jnp.full((16,), 1.0, jnp.float32)

    def mul(epb, gb):
      @pl.loop(0, CHUNK)
      def mul_body(e):
        ev = jnp.full((16,), e, jnp.int32)
        wbits = plsc.load_gather(epb, [idx2, ev])
        w = plsc.bitcast(wbits, jnp.float32)
        dstv = plsc.load_gather(epb, [idx1, ev])
        plsc.addupdate_scatter(hist, [dstv], one16, mask=lane0)
        for j in range(D // 16):
          gb[e, pl.ds(j * 16, 16)] = gb[e, pl.ds(j * 16, 16)] * w

    pend = [None, None]
    gd = [None, None]
    pltpu.async_copy(epack_hbm.at[wid, 0], eps[0], semS).wait()
    gd[0] = pltpu.async_copy(feature_hbm.at[eps[0].at[0]], gbufs[0], sG[0])
    for c in range(CPC):
      cur = c & 1
      nxt = 1 - cur
      if c + 1 < CPC:
        if pend[nxt] is not None:
          pend[nxt].wait()
          pend[nxt] = None
        pltpu.async_copy(epack_hbm.at[wid, c + 1], eps[nxt], semS).wait()
        gd[nxt] = pltpu.async_copy(
            feature_hbm.at[eps[nxt].at[0]], gbufs[nxt], sG[nxt])
      gd[cur].wait()
      mul(eps[cur], gbufs[cur])
      pend[cur] = pltpu.async_copy(
          gbufs[cur], sum_acc.at[eps[cur].at[1]], sA[cur], add=True)
    for bb in range(2):
      if pend[bb] is not None:
        pend[bb].wait()

    plsc.subcore_barrier()
    r = 0
    for n in ZR:
      pltpu.async_copy(sum_acc.at[pl.ds(row0 + r, n)],
                       gbuf.at[pl.ds(0, n)], sem).wait()
      pltpu.async_copy(gbuf.at[pl.ds(0, n)],
                       sums_out.at[cid, pl.ds(row0 + r, n)], sem).wait()
      r += n
    pltpu.async_copy(hist, cnts_out.at[wid], sem).wait()

  return k(feature, epack, z128)


def _tc_combine(feature, sums, cnt2d, W, b2d):
  R = 2000
  grid = N_NODES // R

  def body(feat_ref, sums_ref, cnt_ref, w_ref, b_ref, out_ref):
    s = sums_ref[0]
    for i in range(1, NCALL * NC):
      s = s + sums_ref[i]
    mean = s / jnp.maximum(cnt_ref[...], 1.0)
    w = w_ref[...]
    out_ref[...] = (
        jnp.dot(feat_ref[...], w[:D], preferred_element_type=jnp.float32)
        + jnp.dot(mean, w[D:], preferred_element_type=jnp.float32)
        + b_ref[...]
    )

  return pl.pallas_call(
      body,
      grid=(grid,),
      in_specs=[
          pl.BlockSpec((R, D), lambda i: (i, 0)),
          pl.BlockSpec((NCALL * NC, R, D), lambda i: (0, i, 0)),
          pl.BlockSpec((R, 1), lambda i: (i, 0)),
          pl.BlockSpec((2 * D, D), lambda i: (0, 0)),
          pl.BlockSpec((1, D), lambda i: (0, 0)),
      ],
      out_specs=pl.BlockSpec((R, D), lambda i: (i, 0)),
      out_shape=jax.ShapeDtypeStruct((N_NODES, D), jnp.float32),
  )(feature, sums, cnt2d, W, b2d)


def kernel(feature, relation_indices, relation_weight, W, b):
  ri = relation_indices.astype(jnp.int32)
  npad = EP_TILE - EDGES_PER_TILE
  dst = jnp.pad(ri[0].reshape(NW, EDGES_PER_TILE), ((0, 0), (0, npad)),
                constant_values=PAD_ROW)
  src = jnp.pad(ri[1].reshape(NW, EDGES_PER_TILE), ((0, 0), (0, npad)))
  wtb = lax.bitcast_convert_type(
      relation_weight.reshape(NW, EDGES_PER_TILE), jnp.int32)
  wtb = jnp.pad(wtb, ((0, 0), (0, npad)))
  epack = jnp.stack([src, dst, wtb], axis=1)
  epack = epack.reshape(NW, 3, NCALL * CPC, CHUNK).transpose(0, 2, 1, 3)
  z128 = jnp.zeros((CHUNK, D), jnp.float32)
  sums_l, cnts_l = [], []
  for kcall in range(NCALL):
    s_k, c_k = _sc_aggregate(
        feature, epack[:, kcall * CPC:(kcall + 1) * CPC], z128)
    sums_l.append(s_k)
    cnts_l.append(c_k)
  sums = jnp.concatenate(sums_l, axis=0)
  cnts = sum(c.sum(axis=0) for c in cnts_l)
  cnt2d = cnts[:N_NODES].reshape(N_NODES, 1)
  return _tc_combine(feature, sums, cnt2d, W, b.reshape(1, D))
